# baseline (speedup 1.0000x reference)
import functools

import jax
import jax.numpy as jnp
from jax import lax
from jax.experimental import pallas as pl
from jax.experimental.pallas import tpu as pltpu
from jax.experimental.pallas import tpu_sc as plsc

N = 50000
E = 800000
D_A = 75
D_P = 14

HALF_N = N // 2
ACC_ROWS = 12544
DUMMY_ROW = 12536
CHUNK = 128
NCH = E // CHUNK
ROWS_PER_TILE = ACC_ROWS // 16
PAD_W = 64
UV_W = 128


def _prep_atoms_body(af_ref, w_aa_ref, b_aa_ref, w_uv_ref, b_uv_ref,
                     aa_ref, uv_ref):
    af = af_ref[...]
    aa = jnp.dot(af, w_aa_ref[...], preferred_element_type=jnp.float32)
    aa_ref[...] = jnp.maximum(aa + b_aa_ref[...], 0.0)
    uv = jnp.dot(af, w_uv_ref[...], preferred_element_type=jnp.float32)
    uv_ref[...] = uv + b_uv_ref[...]


def _prep_pairs_body(pf_ref, split_ref, w_pa_ref, b_pa_ref, par_ref):
    pa = jnp.dot(pf_ref[...], w_pa_ref[...], preferred_element_type=jnp.float32)
    pa = jnp.maximum(pa + b_pa_ref[...], 0.0)
    even = split_ref[...] % 2 == 0
    par_ref[...] = jnp.concatenate(
        [jnp.where(even, pa, 0.0), jnp.where(even, 0.0, pa)], axis=-1)


def _final_p_body(aps_ref, pf_ref, w_pp_ref, b_pp_ref, w_top_ref, w_bot_ref,
                  b_po_ref, p_ref):
    pp = jnp.dot(pf_ref[...], w_pp_ref[...], preferred_element_type=jnp.float32)
    pp = jnp.maximum(pp + b_pp_ref[...], 0.0)
    acc = jnp.dot(aps_ref[...], w_top_ref[...], preferred_element_type=jnp.float32)
    acc = acc + jnp.dot(pp, w_bot_ref[...], preferred_element_type=jnp.float32)
    p_ref[...] = jnp.maximum(acc + b_po_ref[...], 0.0)


def _final_a_body(aa_ref, s_ref, w_top_ref, w_bot_ref, b_ao_ref, a_ref):
    acc = jnp.dot(aa_ref[...], w_top_ref[...], preferred_element_type=jnp.float32)
    acc = acc + jnp.dot(s_ref[...], w_bot_ref[...], preferred_element_type=jnp.float32)
    a_ref[...] = jnp.maximum(acc + b_ao_ref[...], 0.0)


CH3 = 32
NCH3 = E // CH3
CPT3 = (NCH3 + 15) // 16


def _seg_sum_body(par_hbm, split_hbm,
                  s_hbm,
                  acc,
                  ia0, d0, ia1, d1, ia2, d2,
                  ls0, ss0, ls1, ss1, ls2, ss2):
    core = lax.axis_index("c")
    sub = lax.axis_index("s")
    base_node = core * HALF_N
    base = sub * CPT3
    cnt = jnp.minimum(CPT3, NCH3 - base)

    IA = (ia0, ia1, ia2)
    DD = (d0, d1, d2)
    LS = (ls0, ls1, ls2)
    SS = (ss0, ss1, ss2)

    zero = jnp.zeros((16,), jnp.float32)

    def _zrow(r, _):
        for k in range(2 * PAD_W // 16):
            d0[r, pl.ds(k * 16, 16)] = zero
        return ()
    lax.fori_loop(0, CH3, _zrow, (), unroll=False)

    rbase = sub * ROWS_PER_TILE
    n_full = ROWS_PER_TILE // CH3
    for b in range(n_full):
        pltpu.sync_copy(d0, acc.at[pl.ds(rbase + b * CH3, CH3)])
    pltpu.sync_copy(d0.at[pl.ds(0, 16)], acc.at[pl.ds(rbase + n_full * CH3, 16)])
    plsc.subcore_barrier()

    def issue_loads(h, s):
        c0 = jnp.minimum(base + h, NCH3 - 1) * CH3
        pltpu.async_copy(split_hbm.at[pl.ds(c0, CH3)], IA[s], LS[s])
        pltpu.async_copy(par_hbm.at[pl.ds(c0, CH3)], DD[s], LS[s])

    def wait_loads(s):
        pltpu.make_async_copy(split_hbm.at[pl.ds(0, CH3)], IA[s], LS[s]).wait()
        pltpu.make_async_copy(par_hbm.at[pl.ds(0, CH3)], DD[s], LS[s]).wait()

    def issue_scatter(s):
        pltpu.async_copy(DD[s], acc.at[IA[s]], SS[s], add=True)

    def wait_scatter(s):
        pltpu.make_async_copy(DD[s], acc.at[IA[s]], SS[s]).wait()

    def clamp(idx_ref, valid_b):
        penalty = jnp.where(valid_b, 0, 2 * N)
        for k in range(CH3 // 16):
            v = idx_ref[pl.ds(k * 16, 16)] - base_node - penalty
            ok = (v >= 0) & (v < HALF_N)
            idx_ref[pl.ds(k * 16, 16)] = jnp.where(ok, v >> 1, DUMMY_ROW)

    def iteration(h, s):
        @pl.when(h >= 1)
        def _():
            wait_scatter((s + 2) % 3)

        @pl.when(h + 2 <= CPT3 - 1)
        def _():
            issue_loads(h + 2, (s + 2) % 3)

        wait_loads(s)
        clamp(IA[s], h < cnt)
        issue_scatter(s)

    issue_loads(0, 0)
    issue_loads(1, 1)

    def outer(h3, _):
        for s in range(3):
            iteration(h3 * 3 + s, s)
        return ()

    n_main = (CPT3 - 2) // 3
    lax.fori_loop(0, n_main, outer, (), unroll=False)
    for h in range(3 * n_main, CPT3):
        iteration(h, h % 3)
    wait_scatter((CPT3 - 1) % 3)

    plsc.subcore_barrier()

    for b in range(n_full):
        pltpu.sync_copy(acc.at[pl.ds(rbase + b * CH3, CH3)], d0)
        pltpu.sync_copy(d0, s_hbm.at[core, pl.ds(rbase + b * CH3, CH3)])
    pltpu.sync_copy(acc.at[pl.ds(rbase + n_full * CH3, 16)], d0.at[pl.ds(0, 16)])
    pltpu.sync_copy(d0.at[pl.ds(0, 16)], s_hbm.at[core, pl.ds(rbase + n_full * CH3, 16)])


CH4 = 64
NCH4 = E // CH4
CPT_G = (NCH4 + 31) // 32


def _gather_combine_body(uv_hbm, i_hbm, j_hbm,
                         aps_hbm,
                         iv0, jv0, ba0, bb0, ob0,
                         iv1, jv1, ba1, bb1, ob1,
                         iv2, jv2, ba2, bb2, ob2,
                         is0, gs0, ss0, is1, gs1, ss1, is2, gs2, ss2):
    core = lax.axis_index("c")
    sub = lax.axis_index("s")
    wid = core * 16 + sub
    base = wid * CPT_G

    IV = (iv0, iv1, iv2)
    JV = (jv0, jv1, jv2)
    BA = (ba0, ba1, ba2)
    BB = (bb0, bb1, bb2)
    OB = (ob0, ob1, ob2)
    IS = (is0, is1, is2)
    GS = (gs0, gs1, gs2)
    SS = (ss0, ss1, ss2)

    def off_of(h):
        return jnp.minimum(base + h, NCH4 - 1) * CH4

    def issue_idx(h, s):
        off = off_of(h)
        pltpu.async_copy(i_hbm.at[pl.ds(off, CH4)], IV[s], IS[s])
        pltpu.async_copy(j_hbm.at[pl.ds(off, CH4)], JV[s], IS[s])

    def wait_idx(s):
        pltpu.make_async_copy(i_hbm.at[pl.ds(0, CH4)], IV[s], IS[s]).wait()
        pltpu.make_async_copy(j_hbm.at[pl.ds(0, CH4)], JV[s], IS[s]).wait()

    def issue_gathers(s):
        pltpu.async_copy(uv_hbm.at[IV[s]], BA[s], GS[s])
        pltpu.async_copy(uv_hbm.at[JV[s]], BB[s], GS[s])

    def wait_gathers(s):
        pltpu.make_async_copy(uv_hbm.at[IV[s]], BA[s], GS[s]).wait()
        pltpu.make_async_copy(uv_hbm.at[JV[s]], BB[s], GS[s]).wait()

    def issue_store(h, s):
        pltpu.async_copy(OB[s], aps_hbm.at[pl.ds(off_of(h), CH4)], SS[s])

    def wait_store(s):
        pltpu.make_async_copy(OB[s], aps_hbm.at[pl.ds(0, CH4)], SS[s]).wait()

    def combine(s):
        ba, bb, ob = BA[s], BB[s], OB[s]

        def _edge(e, _):
            for k in range(PAD_W // 16):
                u_i = ba[e, pl.ds(k * 16, 16)]
                v_i = ba[e, pl.ds(64 + k * 16, 16)]
                u_j = bb[e, pl.ds(k * 16, 16)]
                v_j = bb[e, pl.ds(64 + k * 16, 16)]
                ob[e, pl.ds(k * 16, 16)] = (jnp.maximum(u_i + v_j, 0.0)
                                            + jnp.maximum(u_j + v_i, 0.0))
            return ()
        lax.fori_loop(0, CH4, _edge, (), unroll=False)

    def iteration(h, s):
        @pl.when(h + 2 <= CPT_G - 1)
        def _():
            issue_idx(h + 2, (s + 2) % 3)

        @pl.when(h + 1 <= CPT_G - 1)
        def _():
            wait_idx((s + 1) % 3)
            issue_gathers((s + 1) % 3)

        wait_gathers(s)

        @pl.when(h >= 3)
        def _():
            wait_store(s)

        combine(s)
        issue_store(h, s)

    issue_idx(0, 0)
    issue_idx(1, 1)
    wait_idx(0)
    issue_gathers(0)

    def outer(h3, _):
        for s in range(3):
            iteration(h3 * 3 + s, s)
        return ()

    lax.fori_loop(0, (CPT_G - 1) // 3, outer, (), unroll=False)
    iteration(CPT_G - 1, (CPT_G - 1) % 3)
    for s in range(3):
        wait_store(s)


def _run_seg_sum(par_packed, split32):
    seg = pl.kernel(
        _seg_sum_body,
        out_type=jax.ShapeDtypeStruct((2, ACC_ROWS, 2 * PAD_W), jnp.float32),
        mesh=plsc.VectorSubcoreMesh(core_axis_name="c", subcore_axis_name="s",
                                    num_cores=2, num_subcores=16),
        scratch_types=(
            [pltpu.MemorySpace.VMEM_SHARED((ACC_ROWS, 2 * PAD_W), jnp.float32)]
            + [pltpu.VMEM((CH3,), jnp.int32),
               pltpu.VMEM((CH3, 2 * PAD_W), jnp.float32)] * 3
            + [pltpu.SemaphoreType.DMA] * 6
        ),
        compiler_params=pltpu.CompilerParams(use_tc_tiling_on_sc=True),
    )
    parts = seg(par_packed, split32)
    s0 = parts[0].reshape(2 * ACC_ROWS, PAD_W)[:HALF_N]
    s1 = parts[1].reshape(2 * ACC_ROWS, PAD_W)[:HALF_N]
    return jnp.concatenate([s0, s1], axis=0)


def _run_gather(uv, i_idx, j_idx):
    gat = pl.kernel(
        _gather_combine_body,
        out_type=jax.ShapeDtypeStruct((E, PAD_W), jnp.float32),
        mesh=plsc.VectorSubcoreMesh(core_axis_name="c", subcore_axis_name="s",
                                    num_cores=2, num_subcores=16),
        scratch_types=(
            [pltpu.VMEM((CH4,), jnp.int32),
             pltpu.VMEM((CH4,), jnp.int32),
             pltpu.VMEM((CH4, UV_W), jnp.float32),
             pltpu.VMEM((CH4, UV_W), jnp.float32),
             pltpu.VMEM((CH4, PAD_W), jnp.float32)] * 3
            + [pltpu.SemaphoreType.DMA] * 9
        ),
    )
    return gat(uv, i_idx, j_idx)


def kernel(atom_features, pair_features, pair_split, atom_to_pair,
           W_aa, b_aa, W_pa, b_pa, W_ao, b_ao, W_ap, b_ap, W_pp, b_pp,
           W_po, b_po):
    f32 = jnp.float32

    w_uv = jnp.zeros((D_A, UV_W), f32)
    w_uv = w_uv.at[:, 0:50].set(W_ap[0:D_A, :])
    w_uv = w_uv.at[:, 64:114].set(W_ap[D_A:2 * D_A, :])
    b_uv = jnp.zeros((UV_W,), f32).at[0:50].set(b_ap)

    w_pa_pad = jnp.zeros((D_P, PAD_W), f32).at[:, 0:50].set(W_pa)
    b_pa_pad = jnp.zeros((PAD_W,), f32).at[0:50].set(b_pa)

    w_po_top = jnp.zeros((PAD_W, 50), f32).at[0:50, :].set(W_po[0:50, :])
    w_ao_top = W_ao[0:100, :]
    w_ao_bot = jnp.zeros((PAD_W, 50), f32).at[0:50, :].set(W_ao[100:150, :])

    i_idx = atom_to_pair[:, 0].astype(jnp.int32)
    j_idx = atom_to_pair[:, 1].astype(jnp.int32)
    split32 = pair_split.astype(jnp.int32)

    run_seg_sum = _run_seg_sum
    run_gather = _run_gather

    nb = 50
    blk_n = N // nb
    aa, uv = pl.pallas_call(
        _prep_atoms_body,
        grid=(nb,),
        in_specs=[
            pl.BlockSpec((blk_n, D_A), lambda i: (i, 0)),
            pl.BlockSpec((D_A, 100), lambda i: (0, 0)),
            pl.BlockSpec((100,), lambda i: (0,)),
            pl.BlockSpec((D_A, UV_W), lambda i: (0, 0)),
            pl.BlockSpec((UV_W,), lambda i: (0,)),
        ],
        out_specs=[
            pl.BlockSpec((blk_n, 100), lambda i: (i, 0)),
            pl.BlockSpec((blk_n, UV_W), lambda i: (i, 0)),
        ],
        out_shape=[
            jax.ShapeDtypeStruct((N, 100), f32),
            jax.ShapeDtypeStruct((N, UV_W), f32),
        ],
    )(atom_features, W_aa, b_aa, w_uv, b_uv)

    eb = 200
    blk_e = E // eb
    par = pl.pallas_call(
        _prep_pairs_body,
        grid=(eb,),
        in_specs=[
            pl.BlockSpec((blk_e, D_P), lambda i: (i, 0)),
            pl.BlockSpec((blk_e, 1), lambda i: (i, 0)),
            pl.BlockSpec((D_P, PAD_W), lambda i: (0, 0)),
            pl.BlockSpec((PAD_W,), lambda i: (0,)),
        ],
        out_specs=pl.BlockSpec((blk_e, 2 * PAD_W), lambda i: (i, 0)),
        out_shape=jax.ShapeDtypeStruct((E, 2 * PAD_W), f32),
    )(pair_features, split32.reshape(E, 1), w_pa_pad, b_pa_pad)

    s_sum = run_seg_sum(par, split32)

    aps = run_gather(uv, i_idx, j_idx)

    p_out = pl.pallas_call(
        _final_p_body,
        grid=(eb,),
        in_specs=[
            pl.BlockSpec((blk_e, PAD_W), lambda i: (i, 0)),
            pl.BlockSpec((blk_e, D_P), lambda i: (i, 0)),
            pl.BlockSpec((D_P, 50), lambda i: (0, 0)),
            pl.BlockSpec((50,), lambda i: (0,)),
            pl.BlockSpec((PAD_W, 50), lambda i: (0, 0)),
            pl.BlockSpec((50, 50), lambda i: (0, 0)),
            pl.BlockSpec((50,), lambda i: (0,)),
        ],
        out_specs=pl.BlockSpec((blk_e, 50), lambda i: (i, 0)),
        out_shape=jax.ShapeDtypeStruct((E, 50), f32),
    )(aps, pair_features, W_pp, b_pp, w_po_top, W_po[50:100, :], b_po)

    a_out = pl.pallas_call(
        _final_a_body,
        grid=(nb,),
        in_specs=[
            pl.BlockSpec((blk_n, 100), lambda i: (i, 0)),
            pl.BlockSpec((blk_n, PAD_W), lambda i: (i, 0)),
            pl.BlockSpec((100, 50), lambda i: (0, 0)),
            pl.BlockSpec((PAD_W, 50), lambda i: (0, 0)),
            pl.BlockSpec((50,), lambda i: (0,)),
        ],
        out_specs=pl.BlockSpec((blk_n, 50), lambda i: (i, 0)),
        out_shape=jax.ShapeDtypeStruct((N, 50), f32),
    )(aa, s_sum, w_ao_top, w_ao_bot, b_ao)

    return (a_out, p_out)


# speedup vs baseline: 1.0002x; 1.0002x over previous
import functools

import jax
import jax.numpy as jnp
from jax import lax
from jax.experimental import pallas as pl
from jax.experimental.pallas import tpu as pltpu
from jax.experimental.pallas import tpu_sc as plsc

N = 50000
E = 800000
D_A = 75
D_P = 14

HALF_N = N // 2
ACC_ROWS = 12544
DUMMY_ROW = 12536
CHUNK = 128
NCH = E // CHUNK
ROWS_PER_TILE = ACC_ROWS // 16
PAD_W = 64
UV_W = 128


def _prep_atoms_body(af_ref, w_aa_ref, b_aa_ref, w_uv_ref, b_uv_ref,
                     aa_ref, uv_ref):
    af = af_ref[...]
    aa = jnp.dot(af, w_aa_ref[...], preferred_element_type=jnp.float32)
    aa_ref[...] = jnp.maximum(aa + b_aa_ref[...], 0.0)
    uv = jnp.dot(af, w_uv_ref[...], preferred_element_type=jnp.float32)
    uv_ref[...] = uv + b_uv_ref[...]


def _prep_pairs_body(pf_ref, split_ref, w_pa_ref, b_pa_ref, par_ref):
    pa = jnp.dot(pf_ref[...], w_pa_ref[...], preferred_element_type=jnp.float32)
    pa = jnp.maximum(pa + b_pa_ref[...], 0.0)
    even = split_ref[...] % 2 == 0
    par_ref[...] = jnp.concatenate(
        [jnp.where(even, pa, 0.0), jnp.where(even, 0.0, pa)], axis=-1)


def _final_p_body(aps_ref, pf_ref, w_pp_ref, b_pp_ref, w_top_ref, w_bot_ref,
                  b_po_ref, p_ref):
    pp = jnp.dot(pf_ref[...], w_pp_ref[...], preferred_element_type=jnp.float32)
    pp = jnp.maximum(pp + b_pp_ref[...], 0.0)
    acc = jnp.dot(aps_ref[...], w_top_ref[...], preferred_element_type=jnp.float32)
    acc = acc + jnp.dot(pp, w_bot_ref[...], preferred_element_type=jnp.float32)
    p_ref[...] = jnp.maximum(acc + b_po_ref[...], 0.0)


def _final_a_body(aa_ref, s_ref, w_top_ref, w_bot_ref, b_ao_ref, a_ref):
    acc = jnp.dot(aa_ref[...], w_top_ref[...], preferred_element_type=jnp.float32)
    acc = acc + jnp.dot(s_ref[...], w_bot_ref[...], preferred_element_type=jnp.float32)
    a_ref[...] = jnp.maximum(acc + b_ao_ref[...], 0.0)


CH3 = 32
NCH3 = E // CH3
CPT3 = (NCH3 + 15) // 16


def _seg_sum_body(par_hbm, split_hbm,
                  s_hbm,
                  acc,
                  ia0, d0, ia1, d1, ia2, d2,
                  ls0, ss0, ls1, ss1, ls2, ss2):
    core = lax.axis_index("c")
    sub = lax.axis_index("s")
    base_node = core * HALF_N
    base = sub * CPT3
    cnt = jnp.minimum(CPT3, NCH3 - base)

    IA = (ia0, ia1, ia2)
    DD = (d0, d1, d2)
    LS = (ls0, ls1, ls2)
    SS = (ss0, ss1, ss2)

    zero = jnp.zeros((16,), jnp.float32)

    def _zrow(r, _):
        for k in range(2 * PAD_W // 16):
            d0[r, pl.ds(k * 16, 16)] = zero
        return ()
    lax.fori_loop(0, CH3, _zrow, (), unroll=False)

    rbase = sub * ROWS_PER_TILE
    n_full = ROWS_PER_TILE // CH3
    for b in range(n_full):
        pltpu.sync_copy(d0, acc.at[pl.ds(rbase + b * CH3, CH3)])
    pltpu.sync_copy(d0.at[pl.ds(0, 16)], acc.at[pl.ds(rbase + n_full * CH3, 16)])
    plsc.subcore_barrier()

    def issue_loads(h, s):
        c0 = jnp.minimum(base + h, NCH3 - 1) * CH3
        pltpu.async_copy(split_hbm.at[pl.ds(c0, CH3)], IA[s], LS[s])
        pltpu.async_copy(par_hbm.at[pl.ds(c0, CH3)], DD[s], LS[s])

    def wait_loads(s):
        pltpu.make_async_copy(split_hbm.at[pl.ds(0, CH3)], IA[s], LS[s]).wait()
        pltpu.make_async_copy(par_hbm.at[pl.ds(0, CH3)], DD[s], LS[s]).wait()

    def issue_scatter(s):
        pltpu.async_copy(DD[s], acc.at[IA[s]], SS[s], add=True)

    def wait_scatter(s):
        pltpu.make_async_copy(DD[s], acc.at[IA[s]], SS[s]).wait()

    def clamp(idx_ref, valid_b):
        penalty = jnp.where(valid_b, 0, 2 * N)
        for k in range(CH3 // 16):
            v = idx_ref[pl.ds(k * 16, 16)] - base_node - penalty
            ok = (v >= 0) & (v < HALF_N)
            idx_ref[pl.ds(k * 16, 16)] = jnp.where(ok, v >> 1, DUMMY_ROW)

    def iteration(h, s):
        @pl.when(h >= 1)
        def _():
            wait_scatter((s + 2) % 3)

        @pl.when(h + 2 <= CPT3 - 1)
        def _():
            issue_loads(h + 2, (s + 2) % 3)

        wait_loads(s)
        clamp(IA[s], h < cnt)
        issue_scatter(s)

    issue_loads(0, 0)
    issue_loads(1, 1)

    def outer(h3, _):
        for s in range(3):
            iteration(h3 * 3 + s, s)
        return ()

    n_main = (CPT3 - 2) // 3
    lax.fori_loop(0, n_main, outer, (), unroll=False)
    for h in range(3 * n_main, CPT3):
        iteration(h, h % 3)
    wait_scatter((CPT3 - 1) % 3)

    plsc.subcore_barrier()

    for b in range(n_full):
        pltpu.sync_copy(acc.at[pl.ds(rbase + b * CH3, CH3)], d0)
        pltpu.sync_copy(d0, s_hbm.at[core, pl.ds(rbase + b * CH3, CH3)])
    pltpu.sync_copy(acc.at[pl.ds(rbase + n_full * CH3, 16)], d0.at[pl.ds(0, 16)])
    pltpu.sync_copy(d0.at[pl.ds(0, 16)], s_hbm.at[core, pl.ds(rbase + n_full * CH3, 16)])


CH4 = 64
NCH4 = E // CH4
CPT_G = (NCH4 + 31) // 32


def _gather_combine_body(uv_hbm, i_hbm, j_hbm,
                         aps_hbm,
                         iv0, jv0, ba0, bb0, ob0,
                         iv1, jv1, ba1, bb1, ob1,
                         iv2, jv2, ba2, bb2, ob2,
                         is0, gs0, ss0, is1, gs1, ss1, is2, gs2, ss2):
    core = lax.axis_index("c")
    sub = lax.axis_index("s")
    wid = core * 16 + sub
    base = wid * CPT_G

    IV = (iv0, iv1, iv2)
    JV = (jv0, jv1, jv2)
    BA = (ba0, ba1, ba2)
    BB = (bb0, bb1, bb2)
    OB = (ob0, ob1, ob2)
    IS = (is0, is1, is2)
    GS = (gs0, gs1, gs2)
    SS = (ss0, ss1, ss2)

    zero = jnp.zeros((16,), jnp.float32)

    def _zrow(r, _):
        for s in range(3):
            for k in range(PAD_W // 16):
                OB[s][r, pl.ds(PAD_W + k * 16, 16)] = zero
        return ()
    lax.fori_loop(0, CH4, _zrow, (), unroll=False)

    def off_of(h):
        return jnp.minimum(base + h, NCH4 - 1) * CH4

    def issue_idx(h, s):
        off = off_of(h)
        pltpu.async_copy(i_hbm.at[pl.ds(off, CH4)], IV[s], IS[s])
        pltpu.async_copy(j_hbm.at[pl.ds(off, CH4)], JV[s], IS[s])

    def wait_idx(s):
        pltpu.make_async_copy(i_hbm.at[pl.ds(0, CH4)], IV[s], IS[s]).wait()
        pltpu.make_async_copy(j_hbm.at[pl.ds(0, CH4)], JV[s], IS[s]).wait()

    def issue_gathers(s):
        pltpu.async_copy(uv_hbm.at[IV[s]], BA[s], GS[s])
        pltpu.async_copy(uv_hbm.at[JV[s]], BB[s], GS[s])

    def wait_gathers(s):
        pltpu.make_async_copy(uv_hbm.at[IV[s]], BA[s], GS[s]).wait()
        pltpu.make_async_copy(uv_hbm.at[JV[s]], BB[s], GS[s]).wait()

    def issue_store(h, s):
        pltpu.async_copy(OB[s], aps_hbm.at[pl.ds(off_of(h), CH4)], SS[s])

    def wait_store(s):
        pltpu.make_async_copy(OB[s], aps_hbm.at[pl.ds(0, CH4)], SS[s]).wait()

    def combine(s):
        ba, bb, ob = BA[s], BB[s], OB[s]

        def _edge(e, _):
            for k in range(PAD_W // 16):
                u_i = ba[e, pl.ds(k * 16, 16)]
                v_i = ba[e, pl.ds(64 + k * 16, 16)]
                u_j = bb[e, pl.ds(k * 16, 16)]
                v_j = bb[e, pl.ds(64 + k * 16, 16)]
                ob[e, pl.ds(k * 16, 16)] = (jnp.maximum(u_i + v_j, 0.0)
                                            + jnp.maximum(u_j + v_i, 0.0))
            return ()
        lax.fori_loop(0, CH4, _edge, (), unroll=False)

    def iteration(h, s):
        @pl.when(h + 2 <= CPT_G - 1)
        def _():
            issue_idx(h + 2, (s + 2) % 3)

        @pl.when(h + 1 <= CPT_G - 1)
        def _():
            wait_idx((s + 1) % 3)
            issue_gathers((s + 1) % 3)

        wait_gathers(s)

        @pl.when(h >= 3)
        def _():
            wait_store(s)

        combine(s)
        issue_store(h, s)

    issue_idx(0, 0)
    issue_idx(1, 1)
    wait_idx(0)
    issue_gathers(0)

    def outer(h3, _):
        for s in range(3):
            iteration(h3 * 3 + s, s)
        return ()

    lax.fori_loop(0, (CPT_G - 1) // 3, outer, (), unroll=False)
    iteration(CPT_G - 1, (CPT_G - 1) % 3)
    for s in range(3):
        wait_store(s)


def _run_seg_sum(par_packed, split32):
    seg = pl.kernel(
        _seg_sum_body,
        out_type=jax.ShapeDtypeStruct((2, ACC_ROWS, 2 * PAD_W), jnp.float32),
        mesh=plsc.VectorSubcoreMesh(core_axis_name="c", subcore_axis_name="s",
                                    num_cores=2, num_subcores=16),
        scratch_types=(
            [pltpu.MemorySpace.VMEM_SHARED((ACC_ROWS, 2 * PAD_W), jnp.float32)]
            + [pltpu.VMEM((CH3,), jnp.int32),
               pltpu.VMEM((CH3, 2 * PAD_W), jnp.float32)] * 3
            + [pltpu.SemaphoreType.DMA] * 6
        ),
        compiler_params=pltpu.CompilerParams(use_tc_tiling_on_sc=True),
    )
    parts = seg(par_packed, split32)
    s0 = parts[0].reshape(2 * ACC_ROWS, PAD_W)[:HALF_N]
    s1 = parts[1].reshape(2 * ACC_ROWS, PAD_W)[:HALF_N]
    return jnp.concatenate([s0, s1], axis=0)


def _run_gather(uv, i_idx, j_idx):
    gat = pl.kernel(
        _gather_combine_body,
        out_type=jax.ShapeDtypeStruct((E, UV_W), jnp.float32),
        mesh=plsc.VectorSubcoreMesh(core_axis_name="c", subcore_axis_name="s",
                                    num_cores=2, num_subcores=16),
        scratch_types=(
            [pltpu.VMEM((CH4,), jnp.int32),
             pltpu.VMEM((CH4,), jnp.int32),
             pltpu.VMEM((CH4, UV_W), jnp.float32),
             pltpu.VMEM((CH4, UV_W), jnp.float32),
             pltpu.VMEM((CH4, UV_W), jnp.float32)] * 3
            + [pltpu.SemaphoreType.DMA] * 9
        ),
        compiler_params=pltpu.CompilerParams(use_tc_tiling_on_sc=True),
    )
    return gat(uv, i_idx, j_idx)


def kernel(atom_features, pair_features, pair_split, atom_to_pair,
           W_aa, b_aa, W_pa, b_pa, W_ao, b_ao, W_ap, b_ap, W_pp, b_pp,
           W_po, b_po):
    f32 = jnp.float32

    w_uv = jnp.zeros((D_A, UV_W), f32)
    w_uv = w_uv.at[:, 0:50].set(W_ap[0:D_A, :])
    w_uv = w_uv.at[:, 64:114].set(W_ap[D_A:2 * D_A, :])
    b_uv = jnp.zeros((UV_W,), f32).at[0:50].set(b_ap)

    w_pa_pad = jnp.zeros((D_P, PAD_W), f32).at[:, 0:50].set(W_pa)
    b_pa_pad = jnp.zeros((PAD_W,), f32).at[0:50].set(b_pa)

    w_po_top = jnp.zeros((UV_W, 50), f32).at[0:50, :].set(W_po[0:50, :])
    w_ao_top = W_ao[0:100, :]
    w_ao_bot = jnp.zeros((PAD_W, 50), f32).at[0:50, :].set(W_ao[100:150, :])

    i_idx = atom_to_pair[:, 0].astype(jnp.int32)
    j_idx = atom_to_pair[:, 1].astype(jnp.int32)
    split32 = pair_split.astype(jnp.int32)

    run_seg_sum = _run_seg_sum
    run_gather = _run_gather

    nb = 50
    blk_n = N // nb
    aa, uv = pl.pallas_call(
        _prep_atoms_body,
        grid=(nb,),
        in_specs=[
            pl.BlockSpec((blk_n, D_A), lambda i: (i, 0)),
            pl.BlockSpec((D_A, 100), lambda i: (0, 0)),
            pl.BlockSpec((100,), lambda i: (0,)),
            pl.BlockSpec((D_A, UV_W), lambda i: (0, 0)),
            pl.BlockSpec((UV_W,), lambda i: (0,)),
        ],
        out_specs=[
            pl.BlockSpec((blk_n, 100), lambda i: (i, 0)),
            pl.BlockSpec((blk_n, UV_W), lambda i: (i, 0)),
        ],
        out_shape=[
            jax.ShapeDtypeStruct((N, 100), f32),
            jax.ShapeDtypeStruct((N, UV_W), f32),
        ],
    )(atom_features, W_aa, b_aa, w_uv, b_uv)

    eb = 200
    blk_e = E // eb
    par = pl.pallas_call(
        _prep_pairs_body,
        grid=(eb,),
        in_specs=[
            pl.BlockSpec((blk_e, D_P), lambda i: (i, 0)),
            pl.BlockSpec((blk_e, 1), lambda i: (i, 0)),
            pl.BlockSpec((D_P, PAD_W), lambda i: (0, 0)),
            pl.BlockSpec((PAD_W,), lambda i: (0,)),
        ],
        out_specs=pl.BlockSpec((blk_e, 2 * PAD_W), lambda i: (i, 0)),
        out_shape=jax.ShapeDtypeStruct((E, 2 * PAD_W), f32),
    )(pair_features, split32.reshape(E, 1), w_pa_pad, b_pa_pad)

    s_sum = run_seg_sum(par, split32)

    aps = run_gather(uv, i_idx, j_idx)

    p_out = pl.pallas_call(
        _final_p_body,
        grid=(eb,),
        in_specs=[
            pl.BlockSpec((blk_e, UV_W), lambda i: (i, 0)),
            pl.BlockSpec((blk_e, D_P), lambda i: (i, 0)),
            pl.BlockSpec((D_P, 50), lambda i: (0, 0)),
            pl.BlockSpec((50,), lambda i: (0,)),
            pl.BlockSpec((UV_W, 50), lambda i: (0, 0)),
            pl.BlockSpec((50, 50), lambda i: (0, 0)),
            pl.BlockSpec((50,), lambda i: (0,)),
        ],
        out_specs=pl.BlockSpec((blk_e, 50), lambda i: (i, 0)),
        out_shape=jax.ShapeDtypeStruct((E, 50), f32),
    )(aps, pair_features, W_pp, b_pp, w_po_top, W_po[50:100, :], b_po)

    a_out = pl.pallas_call(
        _final_a_body,
        grid=(nb,),
        in_specs=[
            pl.BlockSpec((blk_n, 100), lambda i: (i, 0)),
            pl.BlockSpec((blk_n, PAD_W), lambda i: (i, 0)),
            pl.BlockSpec((100, 50), lambda i: (0, 0)),
            pl.BlockSpec((PAD_W, 50), lambda i: (0, 0)),
            pl.BlockSpec((50,), lambda i: (0,)),
        ],
        out_specs=pl.BlockSpec((blk_n, 50), lambda i: (i, 0)),
        out_shape=jax.ShapeDtypeStruct((N, 50), f32),
    )(aa, s_sum, w_ao_top, w_ao_bot, b_ao)

    return (a_out, p_out)
